# baseline (speedup 1.0000x reference)
import functools

import jax
import jax.numpy as jnp
from jax import lax
from jax.experimental import pallas as pl
from jax.experimental.pallas import tpu as pltpu
from jax.experimental.pallas import tpu_sc as plsc

S = 64
K = 8
R = 0.05
C = 64
P = 8192

NW = 32
ROWS_PER_W = 2
PIX_PER_W = ROWS_PER_W * S
CAP = 2048
CAND_BUF = CAP + 32
NB = 8
BCAP = 512
COLS_PER_B = S // NB
R2 = R * R
BIG = 1e9
PT_CHUNKS = P // 16


def _kernel_body(px_hbm, py_hbm, pz_hbm, feat_hbm, img_hbm, depth_hbm,
                 px_v, py_v, pz_v,
                 cx_v, cy_v, cz_v, ci_v,
                 bx_v, by_v, bz_v, bi_v,
                 bcnt_v, gidx_v, w_v, rows_v, out_v, depth_v, sem):
    wid = lax.axis_index("s") * 2 + lax.axis_index("c")
    iota = lax.iota(jnp.int32, 16)
    ones_m = iota < 16
    low8 = iota < 8

    pltpu.sync_copy(px_hbm, px_v)
    pltpu.sync_copy(py_hbm, py_v)
    pltpu.sync_copy(pz_hbm, pz_v)

    r0 = wid * ROWS_PER_W
    r0f = r0.astype(jnp.float32)
    y_lo = (r0f + 0.5) * (2.0 / S) - 1.0 - R
    y_hi = (r0f + (ROWS_PER_W - 1) + 0.5) * (2.0 / S) - 1.0 + R

    def scan_pts(i, cnt):
        base = i * 16
        yv = py_v[pl.ds(base, 16)]
        m = (yv >= y_lo) & (yv <= y_hi)
        mi = m.astype(jnp.int32)
        pos = cnt + plsc.cumsum(mi) - 1
        pos = jnp.minimum(pos, CAP - 1)
        xv = px_v[pl.ds(base, 16)]
        zv = pz_v[pl.ds(base, 16)]
        plsc.store_scatter(cx_v, [pos], xv, mask=m)
        plsc.store_scatter(cy_v, [pos], yv, mask=m)
        plsc.store_scatter(cz_v, [pos], zv, mask=m)
        plsc.store_scatter(ci_v, [pos], base + iota, mask=m)
        return jnp.minimum(cnt + jnp.sum(mi), CAP)

    with jax.named_scope("ph_a_compact"):
        cnt = lax.fori_loop(0, PT_CHUNKS, scan_pts, jnp.int32(0))
    plsc.store_scatter(cx_v, [cnt + iota],
                       jnp.full((16,), BIG, jnp.float32), mask=ones_m)
    n_chunks = (cnt + 15) // 16

    def bin_one(b, _):
        bf = b.astype(jnp.float32)
        xlo = (bf * COLS_PER_B + 0.5) * (2.0 / S) - 1.0 - R
        xhi = (bf * COLS_PER_B + (COLS_PER_B - 1) + 0.5) * (2.0 / S) - 1.0 + R

        def bin_scan(j, bcnt):
            base = j * 16
            xv = cx_v[pl.ds(base, 16)]
            m = (xv >= xlo) & (xv <= xhi)
            mi = m.astype(jnp.int32)
            pos = b * BCAP + jnp.minimum(bcnt + plsc.cumsum(mi) - 1, BCAP - 1)
            yv = cy_v[pl.ds(base, 16)]
            zv = cz_v[pl.ds(base, 16)]
            iv = ci_v[pl.ds(base, 16)]
            plsc.store_scatter(bx_v, [pos], xv, mask=m)
            plsc.store_scatter(by_v, [pos], yv, mask=m)
            plsc.store_scatter(bz_v, [pos], zv, mask=m)
            plsc.store_scatter(bi_v, [pos], iv, mask=m)
            return jnp.minimum(bcnt + jnp.sum(mi), BCAP - 16)

        bcnt = lax.fori_loop(0, n_chunks, bin_scan, jnp.int32(0))
        plsc.store_scatter(bx_v, [b * BCAP + bcnt + iota],
                           jnp.full((16,), BIG, jnp.float32), mask=ones_m)
        plsc.store_scatter(bcnt_v, [jnp.full((16,), b, jnp.int32)],
                           jnp.full((16,), bcnt, jnp.int32), mask=iota == 0)
        return _

    with jax.named_scope("ph_a2_bin"):
        lax.fori_loop(0, NB, bin_one, jnp.int32(0))
    bcv = bcnt_v[pl.ds(0, 16)]

    def per_pixel(p, _):
        row_off = p // S
        col = p - row_off * S
        b = col // COLS_PER_B
        cxp = (col.astype(jnp.float32) + 0.5) * (2.0 / S) - 1.0
        cyp = ((r0 + row_off).astype(jnp.float32) + 0.5) * (2.0 / S) - 1.0
        nb_b = jnp.sum(jnp.where(iota == b, bcv, 0))
        nbch = (nb_b + 15) // 16
        bin0 = b * BCAP

        def scan_chunk(j, carry):
            bz, bp = carry
            base = bin0 + j * 16
            xv = bx_v[pl.ds(base, 16)]
            yv = by_v[pl.ds(base, 16)]
            dx = xv - cxp
            dy = yv - cyp
            d2 = dx * dx + dy * dy
            within = d2 < R2
            zv = bz_v[pl.ds(base, 16)]
            zc = jnp.where(within, zv, BIG)
            zd, pd = plsc.sort_key_val(zc, base + iota, descending=True)
            mz = jnp.where(low8, bz, zd)
            mp = jnp.where(low8, bp, pd)
            nz, np_ = plsc.sort_key_val(mz, mp)
            return (nz, np_)

        bz0 = jnp.full((16,), BIG, jnp.float32)
        bp0 = jnp.zeros((16,), jnp.int32)
        bz, bp = lax.fori_loop(0, nbch, scan_chunk, (bz0, bp0))

        valid = low8 & (bz < 100.0)
        safe_p = jnp.where(valid, bp, 0)
        gx = plsc.load_gather(bx_v, [safe_p])
        gy = plsc.load_gather(by_v, [safe_p])
        gz = plsc.load_gather(bz_v, [safe_p])
        gi = plsc.load_gather(bi_v, [safe_p])
        dx = gx - cxp
        dy = gy - cyp
        d2 = dx * dx + dy * dy
        w = jnp.where(valid, 1.0 - d2 / jnp.float32(R2), 0.0)
        den = jnp.sum(w)
        denv = jnp.maximum(jnp.full((16,), den, jnp.float32), 1e-10)
        w = w / denv

        has0 = jnp.sum(jnp.where(valid & (iota == 0), 1, 0)) > 0
        z0 = jnp.sum(jnp.where(iota == 0, gz, 0.0))
        depth = jnp.where(has0, z0, -1.0)
        plsc.store_scatter(depth_v, [jnp.full((16,), p, jnp.int32)],
                           jnp.full((16,), depth, jnp.float32),
                           mask=iota == 0)

        plsc.store_scatter(gidx_v, [p * K + iota],
                           jnp.where(valid, gi, 0), mask=low8)
        plsc.store_scatter(w_v, [p * 16 + iota], w, mask=ones_m)
        return _

    with jax.named_scope("ph_b_topk"):
        lax.fori_loop(0, PIX_PER_W, per_pixel, jnp.int32(0))

    copies = []
    for b in range(8):
        copies.append(pltpu.async_copy(
            feat_hbm.at[gidx_v.at[pl.ds(b * 128, 128)]],
            rows_v.at[pl.ds(b * 128, 128)], sem))
    for cp in copies:
        cp.wait()

    def composite(p, _):
        wv = w_v[pl.ds(p * 16, 16)]
        for cb in range(C // 16):
            acc = jnp.zeros((16,), jnp.float32)
            for k in range(K):
                wk = jnp.sum(jnp.where(iota == k, wv, 0.0))
                acc = acc + wk * rows_v[p * K + k, pl.ds(cb * 16, 16)]
            out_v[pl.ds(p * C + cb * 16, 16)] = acc
        return _

    with jax.named_scope("ph_c_composite"):
        lax.fori_loop(0, PIX_PER_W, composite, jnp.int32(0))

    pltpu.sync_copy(out_v, img_hbm.at[pl.ds(wid * (PIX_PER_W * C), PIX_PER_W * C)])
    pltpu.sync_copy(depth_v, depth_hbm.at[pl.ds(wid * PIX_PER_W, PIX_PER_W)])


@jax.jit
def kernel(points, features):
    px = points[:, 0]
    py = points[:, 1]
    pz = points[:, 2]

    mesh = plsc.VectorSubcoreMesh(core_axis_name="c", subcore_axis_name="s")
    run = functools.partial(
        pl.kernel,
        mesh=mesh,
        compiler_params=pltpu.CompilerParams(
            needs_layout_passes=False, use_tc_tiling_on_sc=False),
        out_type=[
            jax.ShapeDtypeStruct((S * S * C,), jnp.float32),
            jax.ShapeDtypeStruct((S * S,), jnp.float32),
        ],
        scratch_types=[
            pltpu.VMEM((P,), jnp.float32),
            pltpu.VMEM((P,), jnp.float32),
            pltpu.VMEM((P,), jnp.float32),
            pltpu.VMEM((CAND_BUF,), jnp.float32),
            pltpu.VMEM((CAND_BUF,), jnp.float32),
            pltpu.VMEM((CAND_BUF,), jnp.float32),
            pltpu.VMEM((CAND_BUF,), jnp.int32),
            pltpu.VMEM((NB * BCAP,), jnp.float32),
            pltpu.VMEM((NB * BCAP,), jnp.float32),
            pltpu.VMEM((NB * BCAP,), jnp.float32),
            pltpu.VMEM((NB * BCAP,), jnp.int32),
            pltpu.VMEM((16,), jnp.int32),
            pltpu.VMEM((PIX_PER_W * K,), jnp.int32),
            pltpu.VMEM((PIX_PER_W * 16,), jnp.float32),
            pltpu.VMEM((PIX_PER_W * K, C), jnp.float32),
            pltpu.VMEM((PIX_PER_W * C,), jnp.float32),
            pltpu.VMEM((PIX_PER_W,), jnp.float32),
            pltpu.SemaphoreType.DMA,
        ],
    )(_kernel_body)

    img_flat, depth_flat = run(px, py, pz, features)
    images = img_flat.reshape(1, S, S, C)
    depth = depth_flat.reshape(S, S, 1)
    return images, depth


# speedup vs baseline: 1.0976x; 1.0976x over previous
import functools

import jax
import jax.numpy as jnp
from jax import lax
from jax.experimental import pallas as pl
from jax.experimental.pallas import tpu as pltpu
from jax.experimental.pallas import tpu_sc as plsc

S = 64
K = 8
R = 0.05
C = 64
P = 8192

NW = 32
ROWS_PER_W = 2
PIX_PER_W = ROWS_PER_W * S
CAP = 2048
CAND_BUF = CAP + 32
NB = 8
BCAP = 512
COLS_PER_B = S // NB
R2 = R * R
BIG = 1e9
PT_CHUNKS = P // 16


def _kernel_body(px_hbm, py_hbm, pz_hbm, feat_hbm, img_hbm, depth_hbm,
                 px_v, py_v, pz_v,
                 cx_v, cy_v, cz_v, ci_v,
                 bx_v, by_v, bz_v, bi_v,
                 bcnt_v, gidx_v, w_v, rows_v, out_v, depth_v, sem):
    wid = lax.axis_index("s") * 2 + lax.axis_index("c")
    iota = lax.iota(jnp.int32, 16)
    ones_m = iota < 16
    low8 = iota < 8

    pltpu.sync_copy(px_hbm, px_v)
    pltpu.sync_copy(py_hbm, py_v)
    pltpu.sync_copy(pz_hbm, pz_v)

    r0 = wid * ROWS_PER_W
    r0f = r0.astype(jnp.float32)
    y_lo = (r0f + 0.5) * (2.0 / S) - 1.0 - R
    y_hi = (r0f + (ROWS_PER_W - 1) + 0.5) * (2.0 / S) - 1.0 + R

    def scan_pts(i, cnt):
        base = i * 16
        yv = py_v[pl.ds(base, 16)]
        m = (yv >= y_lo) & (yv <= y_hi)
        mi = m.astype(jnp.int32)
        pos = cnt + plsc.cumsum(mi) - 1
        pos = jnp.minimum(pos, CAP - 1)
        xv = px_v[pl.ds(base, 16)]
        zv = pz_v[pl.ds(base, 16)]
        plsc.store_scatter(cx_v, [pos], xv, mask=m)
        plsc.store_scatter(cy_v, [pos], yv, mask=m)
        plsc.store_scatter(cz_v, [pos], zv, mask=m)
        plsc.store_scatter(ci_v, [pos], base + iota, mask=m)
        return jnp.minimum(cnt + jnp.sum(mi), CAP)

    with jax.named_scope("ph_a_compact"):
        cnt = lax.fori_loop(0, PT_CHUNKS, scan_pts, jnp.int32(0))
    plsc.store_scatter(cx_v, [cnt + iota],
                       jnp.full((16,), BIG, jnp.float32), mask=ones_m)
    n_chunks = (cnt + 15) // 16

    def bin_one(b, _):
        bf = b.astype(jnp.float32)
        xlo = (bf * COLS_PER_B + 0.5) * (2.0 / S) - 1.0 - R
        xhi = (bf * COLS_PER_B + (COLS_PER_B - 1) + 0.5) * (2.0 / S) - 1.0 + R

        def bin_scan(j, bcnt):
            base = j * 16
            xv = cx_v[pl.ds(base, 16)]
            m = (xv >= xlo) & (xv <= xhi)
            mi = m.astype(jnp.int32)
            pos = b * BCAP + jnp.minimum(bcnt + plsc.cumsum(mi) - 1, BCAP - 1)
            yv = cy_v[pl.ds(base, 16)]
            zv = cz_v[pl.ds(base, 16)]
            iv = ci_v[pl.ds(base, 16)]
            plsc.store_scatter(bx_v, [pos], xv, mask=m)
            plsc.store_scatter(by_v, [pos], yv, mask=m)
            plsc.store_scatter(bz_v, [pos], zv, mask=m)
            plsc.store_scatter(bi_v, [pos], iv, mask=m)
            return jnp.minimum(bcnt + jnp.sum(mi), BCAP - 16)

        bcnt = lax.fori_loop(0, n_chunks, bin_scan, jnp.int32(0))
        plsc.store_scatter(bx_v, [b * BCAP + bcnt + iota],
                           jnp.full((16,), BIG, jnp.float32), mask=ones_m)
        plsc.store_scatter(bcnt_v, [jnp.full((16,), b, jnp.int32)],
                           jnp.full((16,), bcnt, jnp.int32), mask=iota == 0)
        return _

    with jax.named_scope("ph_a2_bin"):
        lax.fori_loop(0, NB, bin_one, jnp.int32(0))
    bcv = bcnt_v[pl.ds(0, 16)]

    def per_col(col, _):
        b = col // COLS_PER_B
        cxp = (col.astype(jnp.float32) + 0.5) * (2.0 / S) - 1.0
        cyp0 = (r0.astype(jnp.float32) + 0.5) * (2.0 / S) - 1.0
        cyp1 = cyp0 + (2.0 / S)
        nb_b = jnp.sum(jnp.where(iota == b, bcv, 0))
        nbch = (nb_b + 15) // 16
        bin0 = b * BCAP

        def scan_chunk(j, carry):
            bz0, bp0, bz1, bp1 = carry
            base = bin0 + j * 16
            xv = bx_v[pl.ds(base, 16)]
            yv = by_v[pl.ds(base, 16)]
            zv = bz_v[pl.ds(base, 16)]
            dx = xv - cxp
            dxx = dx * dx
            dy0 = yv - cyp0
            dy1 = yv - cyp1
            d20 = dxx + dy0 * dy0
            d21 = dxx + dy1 * dy1
            zc0 = jnp.where(d20 < R2, zv, BIG)
            zc1 = jnp.where(d21 < R2, zv, BIG)
            pv = base + iota
            zd0, pd0 = plsc.sort_key_val(zc0, pv, descending=True)
            zd1, pd1 = plsc.sort_key_val(zc1, pv, descending=True)
            nz0, np0 = plsc.sort_key_val(jnp.where(low8, bz0, zd0),
                                         jnp.where(low8, bp0, pd0))
            nz1, np1 = plsc.sort_key_val(jnp.where(low8, bz1, zd1),
                                         jnp.where(low8, bp1, pd1))
            return (nz0, np0, nz1, np1)

        big0 = jnp.full((16,), BIG, jnp.float32)
        zero0 = jnp.zeros((16,), jnp.int32)
        bz0, bp0, bz1, bp1 = lax.fori_loop(
            0, nbch, scan_chunk, (big0, zero0, big0, zero0))

        for (bzv, bpv, ps, cyp) in ((bz0, bp0, col, cyp0),
                                    (bz1, bp1, col + S, cyp1)):
            valid = low8 & (bzv < 100.0)
            safe_p = jnp.where(valid, bpv, 0)
            gx = plsc.load_gather(bx_v, [safe_p])
            gy = plsc.load_gather(by_v, [safe_p])
            gz = plsc.load_gather(bz_v, [safe_p])
            gi = plsc.load_gather(bi_v, [safe_p])
            dx = gx - cxp
            dy = gy - cyp
            d2 = dx * dx + dy * dy
            w = jnp.where(valid, 1.0 - d2 / jnp.float32(R2), 0.0)
            den = jnp.sum(w)
            denv = jnp.maximum(jnp.full((16,), den, jnp.float32), 1e-10)
            w = w / denv

            has0 = jnp.sum(jnp.where(valid & (iota == 0), 1, 0)) > 0
            z0 = jnp.sum(jnp.where(iota == 0, gz, 0.0))
            depth = jnp.where(has0, z0, -1.0)
            plsc.store_scatter(depth_v, [jnp.full((16,), ps, jnp.int32)],
                               jnp.full((16,), depth, jnp.float32),
                               mask=iota == 0)
            plsc.store_scatter(gidx_v, [ps * K + iota],
                               jnp.where(valid, gi, 0), mask=low8)
            plsc.store_scatter(w_v, [ps * 16 + iota], w, mask=ones_m)
        return _

    with jax.named_scope("ph_b_topk"):
        lax.fori_loop(0, S, per_col, jnp.int32(0))

    copies = []
    for b in range(8):
        copies.append(pltpu.async_copy(
            feat_hbm.at[gidx_v.at[pl.ds(b * 128, 128)]],
            rows_v.at[pl.ds(b * 128, 128)], sem))
    for cp in copies:
        cp.wait()

    def composite(p, _):
        wv = w_v[pl.ds(p * 16, 16)]
        for cb in range(C // 16):
            acc = jnp.zeros((16,), jnp.float32)
            for k in range(K):
                wk = wv[k]
                acc = acc + wk * rows_v[p * K + k, pl.ds(cb * 16, 16)]
            out_v[pl.ds(p * C + cb * 16, 16)] = acc
        return _

    with jax.named_scope("ph_c_composite"):
        lax.fori_loop(0, PIX_PER_W, composite, jnp.int32(0))

    pltpu.sync_copy(out_v, img_hbm.at[pl.ds(wid * (PIX_PER_W * C), PIX_PER_W * C)])
    pltpu.sync_copy(depth_v, depth_hbm.at[pl.ds(wid * PIX_PER_W, PIX_PER_W)])


@jax.jit
def kernel(points, features):
    px = points[:, 0]
    py = points[:, 1]
    pz = points[:, 2]

    mesh = plsc.VectorSubcoreMesh(core_axis_name="c", subcore_axis_name="s")
    run = functools.partial(
        pl.kernel,
        mesh=mesh,
        compiler_params=pltpu.CompilerParams(
            needs_layout_passes=False, use_tc_tiling_on_sc=False),
        out_type=[
            jax.ShapeDtypeStruct((S * S * C,), jnp.float32),
            jax.ShapeDtypeStruct((S * S,), jnp.float32),
        ],
        scratch_types=[
            pltpu.VMEM((P,), jnp.float32),
            pltpu.VMEM((P,), jnp.float32),
            pltpu.VMEM((P,), jnp.float32),
            pltpu.VMEM((CAND_BUF,), jnp.float32),
            pltpu.VMEM((CAND_BUF,), jnp.float32),
            pltpu.VMEM((CAND_BUF,), jnp.float32),
            pltpu.VMEM((CAND_BUF,), jnp.int32),
            pltpu.VMEM((NB * BCAP,), jnp.float32),
            pltpu.VMEM((NB * BCAP,), jnp.float32),
            pltpu.VMEM((NB * BCAP,), jnp.float32),
            pltpu.VMEM((NB * BCAP,), jnp.int32),
            pltpu.VMEM((16,), jnp.int32),
            pltpu.VMEM((PIX_PER_W * K,), jnp.int32),
            pltpu.VMEM((PIX_PER_W * 16,), jnp.float32),
            pltpu.VMEM((PIX_PER_W * K, C), jnp.float32),
            pltpu.VMEM((PIX_PER_W * C,), jnp.float32),
            pltpu.VMEM((PIX_PER_W,), jnp.float32),
            pltpu.SemaphoreType.DMA,
        ],
    )(_kernel_body)

    img_flat, depth_flat = run(px, py, pz, features)
    images = img_flat.reshape(1, S, S, C)
    depth = depth_flat.reshape(S, S, 1)
    return images, depth
